# Initial kernel scaffold; baseline (speedup 1.0000x reference)
import jax
import jax.numpy as jnp
from jax.experimental import pallas as pl


def kernel(words, word_table, pos_table, tt_table, ln_w, ln_b):
    raise NotImplementedError("write your pallas kernel here")


# speedup vs baseline: 1.3618x; 1.3618x over previous
import functools

import jax
import jax.numpy as jnp
from jax import lax
from jax.experimental import pallas as pl
from jax.experimental.pallas import tpu as pltpu
from jax.experimental.pallas import tpu_sc as plsc

B, S, F = 16, 2048, 4
H = 768
VOCAB = 30522
MAX_POS = 8192
EPS = 1e-12

L = 16
KCH = H // L
NW = 32
NROWS = B * S
ROWS_PER_W = NROWS // NW
BATCH_ROWS = 8
PIECES = BATCH_ROWS * F
NBATCH = ROWS_PER_W // BATCH_ROWS
CHUNK = ROWS_PER_W * F
SENT_PIECES = S * F


def _rsqrt(x):
    xi = lax.bitcast_convert_type(x, jnp.int32)
    yi = jnp.int32(0x5F3759DF) - lax.shift_right_logical(xi, 1)
    y = lax.bitcast_convert_type(yi, jnp.float32)
    for _ in range(4):
        y = y * (1.5 - 0.5 * x * y * y)
    return y


def _sc_body(ids_hbm, wt_hbm, pt_hbm, lnw_hbm, lnb_hbm, out_hbm,
             idsbuf, idxw, idxp, wbuf, pbuf, accbuf, outbuf,
             lnwbuf, lnbbuf, sem):
    wid = lax.axis_index("s") * 2 + lax.axis_index("c")
    sent = wid // 2
    half = wid % 2
    iota16 = lax.iota(jnp.int32, 16)

    pltpu.sync_copy(lnw_hbm, lnwbuf)
    pltpu.sync_copy(lnb_hbm, lnbbuf)

    first_half_off = sent * SENT_PIECES
    pltpu.sync_copy(ids_hbm.at[pl.ds(first_half_off, CHUNK)], idsbuf)

    def _count_step(j, cnt):
        v = idsbuf[pl.ds(j * L, L)]
        return cnt + jnp.where(v != 0, 1, 0)

    cnt_v = lax.fori_loop(0, CHUNK // L, _count_step,
                          jnp.zeros((L,), jnp.int32))
    base0 = half * jnp.sum(cnt_v)

    my_off = wid * CHUNK
    pltpu.sync_copy(ids_hbm.at[pl.ds(my_off, CHUNK)], idsbuf)

    def _batch(i, base):
        off = i * PIECES
        ids0 = idsbuf[pl.ds(off, L)]
        ids1 = idsbuf[pl.ds(off + L, L)]
        mi0 = jnp.where(ids0 != 0, 1, 0)
        mi1 = jnp.where(ids1 != 0, 1, 0)
        c0 = plsc.cumsum(mi0)
        c1 = plsc.cumsum(mi1)
        t0 = jnp.sum(mi0)
        t1 = jnp.sum(mi1)
        pos0 = jnp.clip(base + c0 - 1, 0, MAX_POS - 1)
        pos1 = jnp.clip(base + t0 + c1 - 1, 0, MAX_POS - 1)
        idxw[pl.ds(0, L)] = ids0
        idxw[pl.ds(L, L)] = ids1
        idxp[pl.ds(0, L)] = pos0
        idxp[pl.ds(L, L)] = pos1

        dw = pltpu.async_copy(wt_hbm.at[idxw], wbuf, sem)
        dp = pltpu.async_copy(pt_hbm.at[idxp], pbuf, sem)

        zv = jnp.zeros((L,), jnp.float32)

        def _zero(j, _):
            accbuf[pl.ds(j * L, L)] = zv
            return 0

        lax.fori_loop(0, BATCH_ROWS * KCH, _zero, 0)

        dw.wait()
        dp.wait()

        def _piece(p, _):
            def _stat(k, c):
                s, q = c
                x = wbuf[p, pl.ds(k * L, L)] + pbuf[p, pl.ds(k * L, L)]
                return (s + x, q + x * x)

            s_v, q_v = lax.fori_loop(0, KCH, _stat, (zv, zv))
            mu = jnp.sum(s_v) * (1.0 / H)
            var = jnp.sum(q_v) * (1.0 / H) - mu * mu
            rstd = _rsqrt(var + EPS)

            lane = p % L
            sel = iota16 == lane
            m_here = jnp.where(p < L,
                               jnp.sum(jnp.where(sel, mi0, 0)),
                               jnp.sum(jnp.where(sel, mi1, 0)))
            a = rstd * m_here.astype(jnp.float32)
            b = -mu * a
            rowoff = (p // F) * H

            def _norm(k, _):
                x = wbuf[p, pl.ds(k * L, L)] + pbuf[p, pl.ds(k * L, L)]
                plsc.addupdate(accbuf.at[pl.ds(rowoff + k * L, L)],
                               x * a + b)
                return 0

            lax.fori_loop(0, KCH, _norm, 0)
            return 0

        lax.fori_loop(0, PIECES, _piece, 0)

        for r in range(BATCH_ROWS):
            mi = mi0 if r < 4 else mi1
            rr = r % 4
            grp = (iota16 >= rr * F) & (iota16 < rr * F + F)
            cnt = jnp.sum(jnp.where(grp, mi, 0))
            anyv = jnp.where(cnt > 0, 1.0, 0.0)
            inv = jnp.where(cnt <= 1, 1.0,
                            jnp.where(cnt == 2, 0.5,
                                      jnp.where(cnt == 3, 1.0 / 3.0, 0.25)))

            def _fin(k, _, r=r, inv=inv, anyv=anyv):
                o = accbuf[pl.ds(r * H + k * L, L)] * inv
                o = o * lnwbuf[pl.ds(k * L, L)] \
                    + lnbbuf[pl.ds(k * L, L)] * anyv
                outbuf[r, pl.ds(k * L, L)] = o
                return 0

            lax.fori_loop(0, KCH, _fin, 0)

        rowbase = wid * ROWS_PER_W + i * BATCH_ROWS
        pltpu.sync_copy(outbuf, out_hbm.at[pl.ds(rowbase, BATCH_ROWS)])
        return base + t0 + t1

    lax.fori_loop(0, NBATCH, _batch, base0)


def _fold_tt(pt_ref, tt_ref, o_ref):
    o_ref[...] = pt_ref[...] + tt_ref[...]


def kernel(words, word_table, pos_table, tt_table, ln_w, ln_b):
    pos2 = pl.pallas_call(
        _fold_tt,
        grid=(8,),
        in_specs=[
            pl.BlockSpec((MAX_POS // 8, H), lambda i: (i, 0)),
            pl.BlockSpec((1, H), lambda i: (0, 0)),
        ],
        out_specs=pl.BlockSpec((MAX_POS // 8, H), lambda i: (i, 0)),
        out_shape=jax.ShapeDtypeStruct((MAX_POS, H), jnp.float32),
    )(pos_table, tt_table[0:1])

    ids = words.reshape(NROWS * F)

    mesh = plsc.VectorSubcoreMesh(core_axis_name="c", subcore_axis_name="s")
    sc = pl.kernel(
        _sc_body,
        out_type=jax.ShapeDtypeStruct((NROWS, H), jnp.float32),
        mesh=mesh,
        compiler_params=pltpu.CompilerParams(needs_layout_passes=False),
        scratch_types=[
            pltpu.VMEM((CHUNK,), jnp.int32),
            pltpu.VMEM((PIECES,), jnp.int32),
            pltpu.VMEM((PIECES,), jnp.int32),
            pltpu.VMEM((PIECES, H), jnp.float32),
            pltpu.VMEM((PIECES, H), jnp.float32),
            pltpu.VMEM((BATCH_ROWS * H,), jnp.float32),
            pltpu.VMEM((BATCH_ROWS, H), jnp.float32),
            pltpu.VMEM((H,), jnp.float32),
            pltpu.VMEM((H,), jnp.float32),
            pltpu.SemaphoreType.DMA,
        ],
    )
    out = sc(ids, word_table, pos2, ln_w, ln_b)
    return out.reshape(B, S, H)


# speedup vs baseline: 1.4854x; 1.0907x over previous
import jax
import jax.numpy as jnp
from jax import lax
from jax.experimental import pallas as pl
from jax.experimental.pallas import tpu as pltpu
from jax.experimental.pallas import tpu_sc as plsc

B, S, F = 16, 2048, 4
H = 768
VOCAB = 30522
MAX_POS = 8192
EPS = 1e-12

L = 16
KCH = H // L
NW = 32
NROWS = B * S
ROWS_PER_W = NROWS // NW
BATCH_ROWS = 8
PIECES = BATCH_ROWS * F
NBATCH = ROWS_PER_W // BATCH_ROWS
CHUNK = ROWS_PER_W * F
SENT_PIECES = S * F


def _rsqrt(x):
    xi = lax.bitcast_convert_type(x, jnp.int32)
    yi = jnp.int32(0x5F3759DF) - lax.shift_right_logical(xi, 1)
    y = lax.bitcast_convert_type(yi, jnp.float32)
    for _ in range(4):
        y = y * (1.5 - 0.5 * x * y * y)
    return y


def _sc_body(ids_hbm, wt_hbm, pt_hbm, lnw_hbm, lnb_hbm, out_hbm,
             idsbuf, idxw0, idxp0, idxw1, idxp1,
             wbuf0, pbuf0, wbuf1, pbuf1,
             accbuf, xbuf, outbuf0, outbuf1, lnwbuf, lnbbuf,
             gsem0, gsem1, osem0, osem1):
    wid = lax.axis_index("s") * 2 + lax.axis_index("c")
    sent = wid // 2
    half = wid % 2
    iota16 = lax.iota(jnp.int32, 16)
    zv = jnp.zeros((L,), jnp.float32)

    gslot = [(idxw0, idxp0, wbuf0, pbuf0, gsem0),
             (idxw1, idxp1, wbuf1, pbuf1, gsem1)]
    oslot = [(outbuf0, osem0), (outbuf1, osem1)]

    pltpu.sync_copy(lnw_hbm, lnwbuf)
    pltpu.sync_copy(lnb_hbm, lnbbuf)

    first_half_off = sent * SENT_PIECES
    pltpu.sync_copy(ids_hbm.at[pl.ds(first_half_off, CHUNK)], idsbuf)

    def _count_step(j, cnt):
        v = idsbuf[pl.ds(j * L, L)]
        return cnt + jnp.where(v != 0, 1, 0)

    cnt_v = lax.fori_loop(0, CHUNK // L, _count_step,
                          jnp.zeros((L,), jnp.int32), unroll=8)
    base0 = half * jnp.sum(cnt_v)

    my_off = wid * CHUNK
    pltpu.sync_copy(ids_hbm.at[pl.ds(my_off, CHUNK)], idsbuf)

    def _masks(i):
        off = i * PIECES
        ids0 = idsbuf[pl.ds(off, L)]
        ids1 = idsbuf[pl.ds(off + L, L)]
        return (jnp.where(ids0 != 0, 1, 0), jnp.where(ids1 != 0, 1, 0),
                ids0, ids1)

    def _fire(i, base, slot):
        idxw, idxp, wbuf, pbuf, gsem = gslot[slot]
        mi0, mi1, ids0, ids1 = _masks(i)
        c0 = plsc.cumsum(mi0)
        c1 = plsc.cumsum(mi1)
        t0 = jnp.sum(mi0)
        t1 = jnp.sum(mi1)
        pos0 = jnp.clip(base + c0 - 1, 0, MAX_POS - 1)
        pos1 = jnp.clip(base + t0 + c1 - 1, 0, MAX_POS - 1)
        idxw[pl.ds(0, L)] = ids0
        idxw[pl.ds(L, L)] = ids1
        idxp[pl.ds(0, L)] = pos0
        idxp[pl.ds(L, L)] = pos1
        pltpu.async_copy(wt_hbm.at[idxw], wbuf, gsem)
        pltpu.async_copy(pt_hbm.at[idxp], pbuf, gsem)
        return base + t0 + t1

    def _process(i, slot):
        idxw, idxp, wbuf, pbuf, gsem = gslot[slot]
        outbuf, osem = oslot[slot]

        def _zero(j, _):
            accbuf[pl.ds(j * L, L)] = zv
            return 0

        lax.fori_loop(0, BATCH_ROWS * KCH, _zero, 0, unroll=8)

        pltpu.make_async_copy(wt_hbm.at[idxw], wbuf, gsem).wait()
        pltpu.make_async_copy(pt_hbm.at[idxp], pbuf, gsem).wait()

        mi0, mi1, _, _ = _masks(i)

        def _piece(p, _):
            def _stat(k, c):
                s, q = c
                x = wbuf[p, pl.ds(k * L, L)] + pbuf[p, pl.ds(k * L, L)]
                xbuf[pl.ds(k * L, L)] = x
                return (s + x, q + x * x)

            s_v, q_v = lax.fori_loop(0, KCH, _stat, (zv, zv), unroll=8)
            mu = jnp.sum(s_v) * (1.0 / H)
            var = jnp.sum(q_v) * (1.0 / H) - mu * mu
            rstd = _rsqrt(var + EPS)

            lane = p % L
            sel = iota16 == lane
            m_here = jnp.where(p < L,
                               jnp.sum(jnp.where(sel, mi0, 0)),
                               jnp.sum(jnp.where(sel, mi1, 0)))
            a = rstd * m_here.astype(jnp.float32)
            b = -mu * a
            rowoff = (p // F) * H

            def _norm(k, _):
                plsc.addupdate(accbuf.at[pl.ds(rowoff + k * L, L)],
                               xbuf[pl.ds(k * L, L)] * a + b)
                return 0

            lax.fori_loop(0, KCH, _norm, 0, unroll=8)
            return 0

        lax.fori_loop(0, PIECES, _piece, 0)

        @pl.when(i >= 2)
        def _():
            pltpu.make_async_copy(
                outbuf, out_hbm.at[pl.ds(0, BATCH_ROWS)], osem).wait()

        for r in range(BATCH_ROWS):
            mi = mi0 if r < 4 else mi1
            rr = r % 4
            grp = (iota16 >= rr * F) & (iota16 < rr * F + F)
            cnt = jnp.sum(jnp.where(grp, mi, 0))
            anyv = jnp.where(cnt > 0, 1.0, 0.0)
            inv = jnp.where(cnt <= 1, 1.0,
                            jnp.where(cnt == 2, 0.5,
                                      jnp.where(cnt == 3, 1.0 / 3.0, 0.25)))

            def _fin(k, _, r=r, inv=inv, anyv=anyv):
                o = accbuf[pl.ds(r * H + k * L, L)] * inv
                o = o * lnwbuf[pl.ds(k * L, L)] \
                    + lnbbuf[pl.ds(k * L, L)] * anyv
                outbuf[r, pl.ds(k * L, L)] = o
                return 0

            lax.fori_loop(0, KCH, _fin, 0, unroll=6)

        rowbase = wid * ROWS_PER_W + i * BATCH_ROWS
        pltpu.async_copy(outbuf, out_hbm.at[pl.ds(rowbase, BATCH_ROWS)], osem)

    base = _fire(0, base0, 0)

    def _pair(j, base):
        i0 = 2 * j
        base = _fire(i0 + 1, base, 1)
        _process(i0, 0)
        base = _fire(i0 + 2, base, 0)
        _process(i0 + 1, 1)
        return base

    base = lax.fori_loop(0, NBATCH // 2 - 1, _pair, base)
    _fire(NBATCH - 1, base, 1)
    _process(NBATCH - 2, 0)
    _process(NBATCH - 1, 1)

    pltpu.make_async_copy(outbuf0, out_hbm.at[pl.ds(0, BATCH_ROWS)],
                          osem0).wait()
    pltpu.make_async_copy(outbuf1, out_hbm.at[pl.ds(0, BATCH_ROWS)],
                          osem1).wait()


def _fold_tt(pt_ref, tt_ref, o_ref):
    o_ref[...] = pt_ref[...] + tt_ref[...]


def kernel(words, word_table, pos_table, tt_table, ln_w, ln_b):
    pos2 = pl.pallas_call(
        _fold_tt,
        grid=(8,),
        in_specs=[
            pl.BlockSpec((MAX_POS // 8, H), lambda i: (i, 0)),
            pl.BlockSpec((1, H), lambda i: (0, 0)),
        ],
        out_specs=pl.BlockSpec((MAX_POS // 8, H), lambda i: (i, 0)),
        out_shape=jax.ShapeDtypeStruct((MAX_POS, H), jnp.float32),
    )(pos_table, tt_table[0:1])

    ids = words.reshape(NROWS * F)

    mesh = plsc.VectorSubcoreMesh(core_axis_name="c", subcore_axis_name="s")
    sc = pl.kernel(
        _sc_body,
        out_type=jax.ShapeDtypeStruct((NROWS, H), jnp.float32),
        mesh=mesh,
        compiler_params=pltpu.CompilerParams(needs_layout_passes=False),
        scratch_types=[
            pltpu.VMEM((CHUNK,), jnp.int32),
            pltpu.VMEM((PIECES,), jnp.int32),
            pltpu.VMEM((PIECES,), jnp.int32),
            pltpu.VMEM((PIECES,), jnp.int32),
            pltpu.VMEM((PIECES,), jnp.int32),
            pltpu.VMEM((PIECES, H), jnp.float32),
            pltpu.VMEM((PIECES, H), jnp.float32),
            pltpu.VMEM((PIECES, H), jnp.float32),
            pltpu.VMEM((PIECES, H), jnp.float32),
            pltpu.VMEM((BATCH_ROWS * H,), jnp.float32),
            pltpu.VMEM((H,), jnp.float32),
            pltpu.VMEM((BATCH_ROWS, H), jnp.float32),
            pltpu.VMEM((BATCH_ROWS, H), jnp.float32),
            pltpu.VMEM((H,), jnp.float32),
            pltpu.VMEM((H,), jnp.float32),
            pltpu.SemaphoreType.DMA,
            pltpu.SemaphoreType.DMA,
            pltpu.SemaphoreType.DMA,
            pltpu.SemaphoreType.DMA,
        ],
    )
    out = sc(ids, word_table, pos2, ln_w, ln_b)
    return out.reshape(B, S, H)


# speedup vs baseline: 1.8589x; 1.2514x over previous
import jax
import jax.numpy as jnp
from jax import lax
from jax.experimental import pallas as pl
from jax.experimental.pallas import tpu as pltpu
from jax.experimental.pallas import tpu_sc as plsc

B, S, F = 16, 2048, 4
H = 768
VOCAB = 30522
MAX_POS = 8192
EPS = 1e-12

L = 16
KCH = H // L
NW = 32
NROWS = B * S
ROWS_PER_W = NROWS // NW
BATCH_ROWS = 8
PIECES = BATCH_ROWS * F
NBATCH = ROWS_PER_W // BATCH_ROWS
CHUNK = ROWS_PER_W * F
SENT_PIECES = S * F


def _rsqrt(x):
    xi = lax.bitcast_convert_type(x, jnp.int32)
    yi = jnp.int32(0x5F3759DF) - lax.shift_right_logical(xi, 1)
    y = lax.bitcast_convert_type(yi, jnp.float32)
    for _ in range(4):
        y = y * (1.5 - 0.5 * x * y * y)
    return y


def _sc_body(ids_hbm, wt_hbm, pt_hbm, lnw_hbm, lnb_hbm, out_hbm,
             idsbuf, idxw0, idxp0, idxw1, idxp1,
             wbuf0, pbuf0, wbuf1, pbuf1,
             accbuf, sbuf, qbuf, outbuf0, outbuf1, lnwbuf, lnbbuf,
             gsem0, gsem1, osem0, osem1):
    wid = lax.axis_index("s") * 2 + lax.axis_index("c")
    sent = wid // 2
    half = wid % 2
    iota16 = lax.iota(jnp.int32, 16)
    zv = jnp.zeros((L,), jnp.float32)

    gslot = [(idxw0, idxp0, wbuf0, pbuf0, gsem0),
             (idxw1, idxp1, wbuf1, pbuf1, gsem1)]
    oslot = [(outbuf0, osem0), (outbuf1, osem1)]

    pltpu.sync_copy(lnw_hbm, lnwbuf)
    pltpu.sync_copy(lnb_hbm, lnbbuf)

    first_half_off = sent * SENT_PIECES
    pltpu.sync_copy(ids_hbm.at[pl.ds(first_half_off, CHUNK)], idsbuf)

    def _count_step(j, cnt):
        v = idsbuf[pl.ds(j * L, L)]
        return cnt + jnp.where(v != 0, 1, 0)

    cnt_v = lax.fori_loop(0, CHUNK // L, _count_step,
                          jnp.zeros((L,), jnp.int32), unroll=8)
    base0 = half * jnp.sum(cnt_v)

    my_off = wid * CHUNK
    pltpu.sync_copy(ids_hbm.at[pl.ds(my_off, CHUNK)], idsbuf)

    def _masks(i):
        off = i * PIECES
        ids0 = idsbuf[pl.ds(off, L)]
        ids1 = idsbuf[pl.ds(off + L, L)]
        return (jnp.where(ids0 != 0, 1, 0), jnp.where(ids1 != 0, 1, 0),
                ids0, ids1)

    def _fire(i, base, slot):
        idxw, idxp, wbuf, pbuf, gsem = gslot[slot]
        mi0, mi1, ids0, ids1 = _masks(i)
        c0 = plsc.cumsum(mi0)
        c1 = plsc.cumsum(mi1)
        t0 = jnp.sum(mi0)
        t1 = jnp.sum(mi1)
        pos0 = jnp.clip(base + c0 - 1, 0, MAX_POS - 1)
        pos1 = jnp.clip(base + t0 + c1 - 1, 0, MAX_POS - 1)
        idxw[pl.ds(0, L)] = ids0
        idxw[pl.ds(L, L)] = ids1
        idxp[pl.ds(0, L)] = pos0
        idxp[pl.ds(L, L)] = pos1
        pltpu.async_copy(wt_hbm.at[idxw], wbuf, gsem)
        pltpu.async_copy(pt_hbm.at[idxp], pbuf, gsem)
        return base + t0 + t1

    SQG = 17 * L

    def _splat(vec, lane):
        return jnp.take_along_axis(vec, jnp.full((L,), lane, jnp.int32),
                                   axis=0)

    def _process(i, slot):
        idxw, idxp, wbuf, pbuf, gsem = gslot[slot]
        outbuf, osem = oslot[slot]

        def _zero(j, _):
            accbuf[pl.ds(j * L, L)] = zv
            return 0

        lax.fori_loop(0, BATCH_ROWS * KCH, _zero, 0, unroll=8)

        pltpu.make_async_copy(wt_hbm.at[idxw], wbuf, gsem).wait()
        pltpu.make_async_copy(pt_hbm.at[idxp], pbuf, gsem).wait()

        mi0, mi1, _, _ = _masks(i)

        iota17 = iota16 * 17

        def _stats(p, _):
            def _stat(k, c):
                s, q = c
                x = wbuf[p, pl.ds(k * L, L)] + pbuf[p, pl.ds(k * L, L)]
                return (s + x, q + x * x)

            s_v, q_v = lax.fori_loop(0, KCH, _stat, (zv, zv), unroll=8)
            g = p // L
            off = iota17 + (p - g * L + g * SQG)
            plsc.store_scatter(sbuf, [off], s_v)
            plsc.store_scatter(qbuf, [off], q_v)
            return 0

        lax.fori_loop(0, PIECES, _stats, 0)

        abs_ = []
        for g, mi in ((0, mi0), (1, mi1)):
            tot_s = sbuf[pl.ds(g * SQG, L)]
            tot_q = qbuf[pl.ds(g * SQG, L)]
            for c in range(1, L):
                tot_s = tot_s + sbuf[pl.ds(g * SQG + c * 17, L)]
                tot_q = tot_q + qbuf[pl.ds(g * SQG + c * 17, L)]
            mu_v = tot_s * (1.0 / H)
            var_v = tot_q * (1.0 / H) - mu_v * mu_v
            rstd_v = _rsqrt(var_v + EPS)
            a_v = rstd_v * mi.astype(jnp.float32)
            abs_.append((a_v, -mu_v * a_v))

        (a0, b0), (a1, b1) = abs_

        def _piece(p, _):
            pm = p & (L - 1)
            av = jnp.where(p < L, a0, a1)
            bv = jnp.where(p < L, b0, b1)
            a_bc = _splat(av, pm)
            b_bc = _splat(bv, pm)
            rowoff = (p // F) * H

            def _norm(k, _):
                x = wbuf[p, pl.ds(k * L, L)] + pbuf[p, pl.ds(k * L, L)]
                plsc.addupdate(accbuf.at[pl.ds(rowoff + k * L, L)],
                               x * a_bc + b_bc)
                return 0

            lax.fori_loop(0, KCH, _norm, 0, unroll=8)
            return 0

        lax.fori_loop(0, PIECES, _piece, 0)

        @pl.when(i >= 2)
        def _():
            pltpu.make_async_copy(
                outbuf, out_hbm.at[pl.ds(0, BATCH_ROWS)], osem).wait()

        perm1 = iota16 ^ 1
        perm2 = iota16 ^ 2
        invs, anys = [], []
        for mi in (mi0, mi1):
            r1 = mi + jnp.take_along_axis(mi, perm1, axis=0)
            cnt4 = r1 + jnp.take_along_axis(r1, perm2, axis=0)
            invs.append(jnp.where(cnt4 <= 1, 1.0,
                                  jnp.where(cnt4 == 2, 0.5,
                                            jnp.where(cnt4 == 3, 1.0 / 3.0,
                                                      0.25))))
            anys.append(jnp.where(cnt4 > 0, 1.0, 0.0))

        for r in range(BATCH_ROWS):
            g = 0 if r < 4 else 1
            lane = (r % 4) * F
            inv_bc = _splat(invs[g], lane)
            any_bc = _splat(anys[g], lane)

            def _fin(k, _, r=r, inv_bc=inv_bc, any_bc=any_bc):
                o = accbuf[pl.ds(r * H + k * L, L)] * inv_bc
                o = o * lnwbuf[pl.ds(k * L, L)] \
                    + lnbbuf[pl.ds(k * L, L)] * any_bc
                outbuf[r, pl.ds(k * L, L)] = o
                return 0

            lax.fori_loop(0, KCH, _fin, 0, unroll=6)

        rowbase = wid * ROWS_PER_W + i * BATCH_ROWS
        pltpu.async_copy(outbuf, out_hbm.at[pl.ds(rowbase, BATCH_ROWS)], osem)

    base = _fire(0, base0, 0)

    def _pair(j, base):
        i0 = 2 * j
        base = _fire(i0 + 1, base, 1)
        _process(i0, 0)
        base = _fire(i0 + 2, base, 0)
        _process(i0 + 1, 1)
        return base

    base = lax.fori_loop(0, NBATCH // 2 - 1, _pair, base)
    _fire(NBATCH - 1, base, 1)
    _process(NBATCH - 2, 0)
    _process(NBATCH - 1, 1)

    pltpu.make_async_copy(outbuf0, out_hbm.at[pl.ds(0, BATCH_ROWS)],
                          osem0).wait()
    pltpu.make_async_copy(outbuf1, out_hbm.at[pl.ds(0, BATCH_ROWS)],
                          osem1).wait()


def _fold_tt(pt_ref, tt_ref, o_ref):
    o_ref[...] = pt_ref[...] + tt_ref[...]


def kernel(words, word_table, pos_table, tt_table, ln_w, ln_b):
    pos2 = pl.pallas_call(
        _fold_tt,
        grid=(8,),
        in_specs=[
            pl.BlockSpec((MAX_POS // 8, H), lambda i: (i, 0)),
            pl.BlockSpec((1, H), lambda i: (0, 0)),
        ],
        out_specs=pl.BlockSpec((MAX_POS // 8, H), lambda i: (i, 0)),
        out_shape=jax.ShapeDtypeStruct((MAX_POS, H), jnp.float32),
    )(pos_table, tt_table[0:1])

    ids = words.reshape(NROWS * F)

    mesh = plsc.VectorSubcoreMesh(core_axis_name="c", subcore_axis_name="s")
    sc = pl.kernel(
        _sc_body,
        out_type=jax.ShapeDtypeStruct((NROWS, H), jnp.float32),
        mesh=mesh,
        compiler_params=pltpu.CompilerParams(needs_layout_passes=False),
        scratch_types=[
            pltpu.VMEM((CHUNK,), jnp.int32),
            pltpu.VMEM((PIECES,), jnp.int32),
            pltpu.VMEM((PIECES,), jnp.int32),
            pltpu.VMEM((PIECES,), jnp.int32),
            pltpu.VMEM((PIECES,), jnp.int32),
            pltpu.VMEM((PIECES, H), jnp.float32),
            pltpu.VMEM((PIECES, H), jnp.float32),
            pltpu.VMEM((PIECES, H), jnp.float32),
            pltpu.VMEM((PIECES, H), jnp.float32),
            pltpu.VMEM((BATCH_ROWS * H,), jnp.float32),
            pltpu.VMEM((2 * 17 * L,), jnp.float32),
            pltpu.VMEM((2 * 17 * L,), jnp.float32),
            pltpu.VMEM((BATCH_ROWS, H), jnp.float32),
            pltpu.VMEM((BATCH_ROWS, H), jnp.float32),
            pltpu.VMEM((H,), jnp.float32),
            pltpu.VMEM((H,), jnp.float32),
            pltpu.SemaphoreType.DMA,
            pltpu.SemaphoreType.DMA,
            pltpu.SemaphoreType.DMA,
            pltpu.SemaphoreType.DMA,
        ],
    )
    out = sc(ids, word_table, pos2, ln_w, ln_b)
    return out.reshape(B, S, H)
